# baseline (speedup 1.0000x reference)
import functools

import jax
import jax.numpy as jnp
from jax import lax
from jax.experimental import pallas as pl
from jax.experimental.pallas import tpu as pltpu
from jax.experimental.pallas import tpu_sc as plsc

VOCAB_N = 100000
NCORES = 2
NSUB = 16
LANES = 16
HALF = VOCAB_N // NCORES
SLICE = 3136
HPAD = SLICE * NSUB
ROWS = 16384
PRED_W = 50
CAPT_W = 200
ROWS_PER_TILE = ROWS // NSUB
PRED_RCHUNK = 128
CAPT_RCHUNK = 32
CHUNK = 6400

_mesh = plsc.VectorSubcoreMesh(core_axis_name="c", subcore_axis_name="s")


@functools.partial(
    pl.kernel,
    out_type=jax.ShapeDtypeStruct((NCORES * NSUB, 2, LANES), jnp.int32),
    mesh=_mesh,
    scratch_types=[
        pltpu.VMEM((HPAD,), jnp.int32),
        pltpu.VMEM((2, PRED_RCHUNK, PRED_W), jnp.int32),
        pltpu.VMEM((2, CAPT_RCHUNK, CAPT_W), jnp.int32),
        pltpu.VMEM((2, LANES), jnp.int32),
        pltpu.VMEM_SHARED((NSUB, HPAD), jnp.int32),
        pltpu.SemaphoreType.DMA,
        pltpu.SemaphoreType.DMA,
        pltpu.SemaphoreType.DMA,
    ],
    compiler_params=pltpu.CompilerParams(use_tc_tiling_on_sc=False,
                                         needs_layout_passes=False),
)
def _vocab_usage_sc(preds_hbm, capts_hbm, out_hbm,
                    pres, pbuf, tbuf, cbuf, shared, sem_a, sem_b, sem_m):
    core = lax.axis_index("c")
    sub = lax.axis_index("s")
    wid = core * NSUB + sub
    base = core * HALF
    zeros16 = jnp.zeros((LANES,), jnp.int32)
    ones16 = jnp.ones((LANES,), jnp.int32)
    sl_start = sub * SLICE

    def _zero_pres():
        @plsc.parallel_loop(0, HPAD // LANES, unroll=8)
        def _z(i):
            pres[pl.ds(i * LANES, LANES)] = zeros16

    def _scatter(src_hbm, buf, rchunk, width):
        tile_base = sub * ROWS_PER_TILE
        nch = ROWS_PER_TILE // rchunk
        cols = list(range(0, width - LANES + 1, LANES))
        if cols[-1] + LANES < width:
            cols.append(width - LANES)
        sems = (sem_a, sem_b)

        def _process(b, r):
            for c in cols:
                tok = buf[b, r, pl.ds(c, LANES)]
                loc = tok - base
                msk = loc.astype(jnp.uint32) < jnp.uint32(HALF)
                plsc.store_scatter(pres, [loc], ones16, mask=msk)

        pltpu.async_copy(src_hbm.at[pl.ds(tile_base, rchunk)],
                         buf.at[0], sems[0])

        def _pair(p, carry):
            ch = 2 * p
            for b in (0, 1):
                nxt = ch + b + 1

                @pl.when(nxt < nch)
                def _():
                    pltpu.async_copy(
                        src_hbm.at[pl.ds(tile_base + nxt * rchunk, rchunk)],
                        buf.at[1 - b], sems[1 - b])
                pltpu.make_async_copy(src_hbm.at[pl.ds(tile_base, rchunk)],
                                      buf.at[b], sems[b]).wait()

                @plsc.parallel_loop(0, rchunk, unroll=2)
                def _v(r):
                    _process(b, r)
            return carry
        lax.fori_loop(0, nch // 2, _pair, 0)

    def _merge_count(inp):
        descs = [pltpu.async_copy(shared.at[t, pl.ds(sl_start, SLICE)],
                                  pres.at[pl.ds(t * SLICE, SLICE)], sem_m)
                 for t in range(NSUB)]
        for d in descs:
            d.wait()

        @plsc.parallel_loop(0, SLICE // LANES, unroll=4, carry=zeros16)
        def _cnt(j, cv):
            acc = pres[pl.ds(j * LANES, LANES)]
            for t in range(1, NSUB):
                acc = acc | pres[pl.ds(t * SLICE + j * LANES, LANES)]
            return cv + (acc != 0).astype(jnp.int32)
        cbuf[inp] = _cnt

    _zero_pres()
    _scatter(preds_hbm, pbuf, PRED_RCHUNK, PRED_W)
    pltpu.sync_copy(pres, shared.at[sub])
    plsc.subcore_barrier()
    _merge_count(0)
    plsc.subcore_barrier()
    _zero_pres()
    _scatter(capts_hbm, tbuf, CAPT_RCHUNK, CAPT_W)
    pltpu.sync_copy(pres, shared.at[sub])
    plsc.subcore_barrier()
    _merge_count(1)

    pltpu.sync_copy(cbuf, out_hbm.at[wid])


def kernel(preds, captions):
    parts = _vocab_usage_sc(preds, captions)
    n_pred = parts[:, 0, :].sum().astype(jnp.float32)
    n_capt = parts[:, 1, :].sum().astype(jnp.float32)
    return jnp.where(n_capt > 0, n_pred / jnp.maximum(n_capt, 1.0),
                     jnp.float32(0.0))


# speedup vs baseline: 1.4235x; 1.4235x over previous
import functools

import jax
import jax.numpy as jnp
from jax import lax
from jax.experimental import pallas as pl
from jax.experimental.pallas import tpu as pltpu
from jax.experimental.pallas import tpu_sc as plsc

VOCAB_N = 100000
NCORES = 2
NSUB = 16
LANES = 16
HALF = VOCAB_N // NCORES
SLICE = 3136
HPAD = SLICE * NSUB
N_PRED = 16384 * 50
N_CAPT = 16384 * 200
PRED_PER_TILE = N_PRED // NSUB
CAPT_PER_TILE = N_CAPT // NSUB
CHUNK = 6400

_mesh = plsc.VectorSubcoreMesh(core_axis_name="c", subcore_axis_name="s")


@functools.partial(
    pl.kernel,
    out_type=jax.ShapeDtypeStruct((NCORES * NSUB, 2, LANES), jnp.int32),
    mesh=_mesh,
    scratch_types=[
        pltpu.VMEM((HPAD,), jnp.int32),
        pltpu.VMEM((2, CHUNK), jnp.int32),
        pltpu.VMEM((2, LANES), jnp.int32),
        pltpu.VMEM_SHARED((NSUB, HPAD), jnp.int32),
        pltpu.SemaphoreType.DMA,
        pltpu.SemaphoreType.DMA,
        pltpu.SemaphoreType.DMA,
    ],
    compiler_params=pltpu.CompilerParams(use_tc_tiling_on_sc=False,
                                         needs_layout_passes=False),
)
def _vocab_usage_sc(preds_hbm, capts_hbm, out_hbm,
                    pres, tbuf, cbuf, shared, sem_a, sem_b, sem_m):
    core = lax.axis_index("c")
    sub = lax.axis_index("s")
    wid = core * NSUB + sub
    base = core * HALF
    zeros16 = jnp.zeros((LANES,), jnp.int32)
    ones16 = jnp.ones((LANES,), jnp.int32)
    sl_start = sub * SLICE

    def _zero_pres():
        @plsc.parallel_loop(0, HPAD // LANES, unroll=8)
        def _z(i):
            pres[pl.ds(i * LANES, LANES)] = zeros16

    def _scatter(src_hbm, per_tile):
        tile_base = sub * per_tile
        nch = per_tile // CHUNK
        sems = (sem_a, sem_b)
        desc = [None, None]
        desc[0] = pltpu.async_copy(src_hbm.at[pl.ds(tile_base, CHUNK)],
                                   tbuf.at[0], sems[0])
        for ch in range(nch):
            b = ch % 2
            if ch + 1 < nch:
                nb = (ch + 1) % 2
                desc[nb] = pltpu.async_copy(
                    src_hbm.at[pl.ds(tile_base + (ch + 1) * CHUNK, CHUNK)],
                    tbuf.at[nb], sems[nb])
            desc[b].wait()

            @plsc.parallel_loop(0, CHUNK // LANES, unroll=8)
            def _v(i):
                tok = tbuf[b, pl.ds(i * LANES, LANES)]
                loc = tok - base
                msk = loc.astype(jnp.uint32) < jnp.uint32(HALF)
                plsc.store_scatter(pres, [loc], ones16, mask=msk)

    def _merge_count(inp):
        descs = [pltpu.async_copy(shared.at[t, pl.ds(sl_start, SLICE)],
                                  pres.at[pl.ds(t * SLICE, SLICE)], sem_m)
                 for t in range(NSUB)]
        for d in descs:
            d.wait()

        @plsc.parallel_loop(0, SLICE // LANES, unroll=4, carry=zeros16)
        def _cnt(j, cv):
            acc = pres[pl.ds(j * LANES, LANES)]
            for t in range(1, NSUB):
                acc = acc | pres[pl.ds(t * SLICE + j * LANES, LANES)]
            return cv + (acc != 0).astype(jnp.int32)
        cbuf[inp] = _cnt

    _zero_pres()
    _scatter(preds_hbm, PRED_PER_TILE)
    pltpu.sync_copy(pres, shared.at[sub])
    plsc.subcore_barrier()
    _merge_count(0)
    plsc.subcore_barrier()
    _zero_pres()
    _scatter(capts_hbm, CAPT_PER_TILE)
    pltpu.sync_copy(pres, shared.at[sub])
    plsc.subcore_barrier()
    _merge_count(1)

    pltpu.sync_copy(cbuf, out_hbm.at[wid])


def kernel(preds, captions):
    parts = _vocab_usage_sc(preds.T.reshape(-1), captions.T.reshape(-1))
    n_pred = parts[:, 0, :].sum().astype(jnp.float32)
    n_capt = parts[:, 1, :].sum().astype(jnp.float32)
    return jnp.where(n_capt > 0, n_pred / jnp.maximum(n_capt, 1.0),
                     jnp.float32(0.0))


# speedup vs baseline: 1.8461x; 1.2969x over previous
import functools

import jax
import jax.numpy as jnp
from jax import lax
from jax.experimental import pallas as pl
from jax.experimental.pallas import tpu as pltpu
from jax.experimental.pallas import tpu_sc as plsc

VOCAB_N = 100000
NCORES = 2
NSUB = 16
LANES = 16
HALF = VOCAB_N // NCORES
BMAP = 1792
MSLICE = BMAP // NSUB
HPAD = 32 * BMAP
N_PRED = 16384 * 50
N_CAPT = 16384 * 200
PRED_PER_TILE = N_PRED // NSUB
CAPT_PER_TILE = N_CAPT // NSUB
CHUNK = 6400

_mesh = plsc.VectorSubcoreMesh(core_axis_name="c", subcore_axis_name="s")


@functools.partial(
    pl.kernel,
    out_type=jax.ShapeDtypeStruct((NCORES * NSUB, 2, LANES), jnp.int32),
    mesh=_mesh,
    scratch_types=[
        pltpu.VMEM((HPAD,), jnp.int32),
        pltpu.VMEM((2, CHUNK), jnp.int32),
        pltpu.VMEM((2, LANES), jnp.int32),
        pltpu.VMEM_SHARED((NSUB, BMAP), jnp.int32),
        pltpu.SemaphoreType.DMA,
        pltpu.SemaphoreType.DMA,
        pltpu.SemaphoreType.DMA,
    ],
    compiler_params=pltpu.CompilerParams(use_tc_tiling_on_sc=False,
                                         needs_layout_passes=False),
)
def _vocab_usage_sc(preds_hbm, capts_hbm, out_hbm,
                    pres, tbuf, cbuf, shared, sem_a, sem_b, sem_m):
    core = lax.axis_index("c")
    sub = lax.axis_index("s")
    wid = core * NSUB + sub
    base = core * HALF
    zeros16 = jnp.zeros((LANES,), jnp.int32)
    ones16 = jnp.ones((LANES,), jnp.int32)

    def _zero_pres():
        @plsc.parallel_loop(0, HPAD // LANES, unroll=8)
        def _z(i):
            pres[pl.ds(i * LANES, LANES)] = zeros16

    def _scatter(src_hbm, per_tile):
        tile_base = sub * per_tile
        nch = per_tile // CHUNK
        sems = (sem_a, sem_b)
        desc = [None, None]
        desc[0] = pltpu.async_copy(src_hbm.at[pl.ds(tile_base, CHUNK)],
                                   tbuf.at[0], sems[0])
        for ch in range(nch):
            b = ch % 2
            if ch + 1 < nch:
                nb = (ch + 1) % 2
                desc[nb] = pltpu.async_copy(
                    src_hbm.at[pl.ds(tile_base + (ch + 1) * CHUNK, CHUNK)],
                    tbuf.at[nb], sems[nb])
            desc[b].wait()

            @plsc.parallel_loop(0, CHUNK // LANES, unroll=8)
            def _v(i):
                tok = tbuf[b, pl.ds(i * LANES, LANES)]
                loc = tok - base
                msk = loc.astype(jnp.uint32) < jnp.uint32(HALF)
                plsc.store_scatter(pres, [loc], ones16, mask=msk)

    def _pack():
        @plsc.parallel_loop(0, BMAP // LANES, unroll=2)
        def _p(i):
            acc = pres[pl.ds(i * LANES, LANES)]
            for j in range(1, 32):
                acc = acc | (pres[pl.ds(j * BMAP + i * LANES, LANES)] << j)
            tbuf[0, pl.ds(i * LANES, LANES)] = acc

    def _merge_count(inp):
        descs = [pltpu.async_copy(shared.at[t, pl.ds(sub * MSLICE, MSLICE)],
                                  tbuf.at[1, pl.ds(t * MSLICE, MSLICE)], sem_m)
                 for t in range(NSUB)]
        for d in descs:
            d.wait()

        @plsc.parallel_loop(0, MSLICE // LANES, unroll=1, carry=zeros16)
        def _cnt(j, cv):
            acc = tbuf[1, pl.ds(j * LANES, LANES)]
            for t in range(1, NSUB):
                acc = acc | tbuf[1, pl.ds(t * MSLICE + j * LANES, LANES)]
            u = plsc.bitcast(acc, jnp.uint32)
            u = u - ((u >> jnp.uint32(1)) & jnp.uint32(0x55555555))
            u = ((u & jnp.uint32(0x33333333))
                 + ((u >> jnp.uint32(2)) & jnp.uint32(0x33333333)))
            u = (u + (u >> jnp.uint32(4))) & jnp.uint32(0x0F0F0F0F)
            u = (u * jnp.uint32(0x01010101)) >> jnp.uint32(24)
            return cv + plsc.bitcast(u, jnp.int32)
        cbuf[inp] = _cnt

    def _one_input(inp, src_hbm, per_tile):
        _zero_pres()
        _scatter(src_hbm, per_tile)
        _pack()
        pltpu.sync_copy(tbuf.at[0, pl.ds(0, BMAP)], shared.at[sub])
        plsc.subcore_barrier()
        _merge_count(inp)
        plsc.subcore_barrier()

    _one_input(0, preds_hbm, PRED_PER_TILE)
    _one_input(1, capts_hbm, CAPT_PER_TILE)

    pltpu.sync_copy(cbuf, out_hbm.at[wid])


def kernel(preds, captions):
    pf = preds.T.reshape(-1)
    cf = captions.reshape(128, 128, 25, 8).transpose(2, 0, 3, 1).reshape(-1)
    parts = _vocab_usage_sc(pf, cf)
    n_pred = parts[:, 0, :].sum().astype(jnp.float32)
    n_capt = parts[:, 1, :].sum().astype(jnp.float32)
    return jnp.where(n_capt > 0, n_pred / jnp.maximum(n_capt, 1.0),
                     jnp.float32(0.0))


# speedup vs baseline: 2.0965x; 1.1356x over previous
import functools

import jax
import jax.numpy as jnp
from jax import lax
from jax.experimental import pallas as pl
from jax.experimental.pallas import tpu as pltpu
from jax.experimental.pallas import tpu_sc as plsc

VOCAB_N = 100000
NCORES = 2
NSUB = 16
LANES = 16
HALF = VOCAB_N // NCORES
BMAP = 1792
MSLICE = BMAP // NSUB
HPAD = 32 * BMAP
N_PRED = 16384 * 50
N_CAPT = 16384 * 200
PRED_PER_TILE = N_PRED // NSUB
CAPT_PER_TILE = N_CAPT // NSUB
CHUNK = 12800

_mesh = plsc.VectorSubcoreMesh(core_axis_name="c", subcore_axis_name="s")


@functools.partial(
    pl.kernel,
    out_type=jax.ShapeDtypeStruct((NCORES * NSUB, 2, LANES), jnp.int32),
    mesh=_mesh,
    scratch_types=[
        pltpu.VMEM((HPAD,), jnp.int32),
        pltpu.VMEM((2, CHUNK), jnp.int32),
        pltpu.VMEM((BMAP,), jnp.int32),
        pltpu.VMEM((BMAP,), jnp.int32),
        pltpu.VMEM((2, LANES), jnp.int32),
        pltpu.VMEM_SHARED((NSUB, BMAP), jnp.int32),
        pltpu.SemaphoreType.DMA,
        pltpu.SemaphoreType.DMA,
        pltpu.SemaphoreType.DMA,
    ],
    compiler_params=pltpu.CompilerParams(use_tc_tiling_on_sc=False,
                                         needs_layout_passes=False),
)
def _vocab_usage_sc(preds_hbm, capts_hbm, out_hbm,
                    pres, tbuf, pkbuf, mbuf, cbuf, shared,
                    sem_a, sem_b, sem_m):
    core = lax.axis_index("c")
    sub = lax.axis_index("s")
    wid = core * NSUB + sub
    base = core * HALF
    zeros16 = jnp.zeros((LANES,), jnp.int32)
    ones16 = jnp.ones((LANES,), jnp.int32)
    sems = (sem_a, sem_b)

    def _zero_pres():
        @plsc.parallel_loop(0, HPAD // LANES, unroll=8)
        def _z(i):
            pres[pl.ds(i * LANES, LANES)] = zeros16

    def _prefetch(src_hbm, per_tile):
        pltpu.async_copy(src_hbm.at[pl.ds(sub * per_tile, CHUNK)],
                         tbuf.at[0], sems[0])

    def _scatter(src_hbm, per_tile):
        tile_base = sub * per_tile
        nch = per_tile // CHUNK
        for ch in range(nch):
            b = ch % 2
            if ch + 1 < nch:
                nb = (ch + 1) % 2
                pltpu.async_copy(
                    src_hbm.at[pl.ds(tile_base + (ch + 1) * CHUNK, CHUNK)],
                    tbuf.at[nb], sems[nb])
            pltpu.make_async_copy(src_hbm.at[pl.ds(tile_base, CHUNK)],
                                  tbuf.at[b], sems[b]).wait()

            @plsc.parallel_loop(0, CHUNK // LANES, unroll=16)
            def _v(i):
                tok = tbuf[b, pl.ds(i * LANES, LANES)]
                loc = tok - base
                msk = loc.astype(jnp.uint32) < jnp.uint32(HALF)
                plsc.store_scatter(pres, [loc], ones16, mask=msk)

    def _pack():
        @plsc.parallel_loop(0, BMAP // LANES, unroll=2)
        def _p(i):
            acc = pres[pl.ds(i * LANES, LANES)]
            for j in range(1, 32):
                acc = acc | (pres[pl.ds(j * BMAP + i * LANES, LANES)] << j)
            pkbuf[pl.ds(i * LANES, LANES)] = acc

    def _merge_count(inp):
        descs = [pltpu.async_copy(shared.at[t, pl.ds(sub * MSLICE, MSLICE)],
                                  mbuf.at[pl.ds(t * MSLICE, MSLICE)], sem_m)
                 for t in range(NSUB)]
        for d in descs:
            d.wait()

        @plsc.parallel_loop(0, MSLICE // LANES, unroll=1, carry=zeros16)
        def _cnt(j, cv):
            acc = mbuf[pl.ds(j * LANES, LANES)]
            for t in range(1, NSUB):
                acc = acc | mbuf[pl.ds(t * MSLICE + j * LANES, LANES)]
            u = plsc.bitcast(acc, jnp.uint32)
            u = u - ((u >> jnp.uint32(1)) & jnp.uint32(0x55555555))
            u = ((u & jnp.uint32(0x33333333))
                 + ((u >> jnp.uint32(2)) & jnp.uint32(0x33333333)))
            u = (u + (u >> jnp.uint32(4))) & jnp.uint32(0x0F0F0F0F)
            u = (u * jnp.uint32(0x01010101)) >> jnp.uint32(24)
            return cv + plsc.bitcast(u, jnp.int32)
        cbuf[inp] = _cnt

    _zero_pres()
    _prefetch(preds_hbm, PRED_PER_TILE)
    _scatter(preds_hbm, PRED_PER_TILE)
    _pack()
    pltpu.sync_copy(pkbuf, shared.at[sub])
    _zero_pres()
    _prefetch(capts_hbm, CAPT_PER_TILE)
    plsc.subcore_barrier()
    _merge_count(0)
    plsc.subcore_barrier()
    _scatter(capts_hbm, CAPT_PER_TILE)
    _pack()
    pltpu.sync_copy(pkbuf, shared.at[sub])
    plsc.subcore_barrier()
    _merge_count(1)

    pltpu.sync_copy(cbuf, out_hbm.at[wid])


def kernel(preds, captions):
    pf = preds.T.reshape(-1)
    cf = captions.reshape(128, 128, 25, 8).transpose(2, 0, 3, 1).reshape(-1)
    parts = _vocab_usage_sc(pf, cf)
    n_pred = parts[:, 0, :].sum().astype(jnp.float32)
    n_capt = parts[:, 1, :].sum().astype(jnp.float32)
    return jnp.where(n_capt > 0, n_pred / jnp.maximum(n_capt, 1.0),
                     jnp.float32(0.0))
